# baseline (speedup 1.0000x reference)
import functools

import jax
import jax.numpy as jnp
from jax import lax
from jax.experimental import pallas as pl
from jax.experimental.pallas import tpu as pltpu
from jax.experimental.pallas import tpu_sc as plsc

_N = 10000
_H = 256
_NACC = 10240
_F32 = jnp.float32


def _proj_body(x_ref, w_ref, b_ref, wl_ref, o_ref, hw_ref):
    h = jnp.maximum(
        jnp.dot(x_ref[...], w_ref[...], preferred_element_type=_F32)
        + b_ref[...], 0.0)
    o_ref[...] = h
    hw = jnp.dot(h, wl_ref[...], preferred_element_type=_F32)
    hw_ref[0] = hw[:, :128]
    hw_ref[1] = hw[:, 128:]


def _proj(x, Wp, bp, Wl0):
    return pl.pallas_call(
        _proj_body,
        grid=(10,),
        in_specs=[
            pl.BlockSpec((1000, 128), lambda i: (i, 0)),
            pl.BlockSpec((128, _H), lambda i: (0, 0)),
            pl.BlockSpec((1, _H), lambda i: (0, 0)),
            pl.BlockSpec((_H, _H), lambda i: (0, 0)),
        ],
        out_specs=[
            pl.BlockSpec((1000, _H), lambda i: (i, 0)),
            pl.BlockSpec((2, 1000, 128), lambda i: (0, i, 0)),
        ],
        out_shape=[
            jax.ShapeDtypeStruct((_N, _H), _F32),
            jax.ShapeDtypeStruct((2, _N, 128), _F32),
        ],
    )(x, Wp, bp.reshape(1, _H), Wl0)


def _post_body(agg_ref, dacc_ref, h_ref, wr_ref, bl_ref, g_ref, be_ref,
               wn_ref, o_ref, hw_ref):
    mean_cat = jnp.concatenate([agg_ref[0], agg_ref[1]], axis=-1)
    deg = dacc_ref[0, :, 0:1] + dacc_ref[1, :, 0:1]
    m = jnp.maximum(deg, 1.0)
    h = h_ref[...]
    z = mean_cat / m + bl_ref[...] + jnp.dot(
        h, wr_ref[...], preferred_element_type=_F32)
    mu = jnp.mean(z, axis=-1, keepdims=True)
    zc = z - mu
    var = jnp.mean(zc * zc, axis=-1, keepdims=True)
    zn = zc * lax.rsqrt(var + 1e-5) * g_ref[...] + be_ref[...]
    h_new = jnp.maximum(zn, 0.0) + h
    o_ref[...] = h_new
    hw = jnp.dot(h_new, wn_ref[...], preferred_element_type=_F32)
    if hw_ref.shape[0] == 2:
        hw_ref[0] = hw[:, :128]
        hw_ref[1] = hw[:, 128:]
    else:
        hw_ref[...] = hw


def _post(agg, dacc, h, Wr, bl, g, be, Wnext):
    wide = Wnext.shape[1] == _H
    if wide:
        hw_spec = pl.BlockSpec((2, 1000, 128), lambda r: (0, r, 0))
        hw_shape = jax.ShapeDtypeStruct((2, _N, 128), _F32)
    else:
        hw_spec = pl.BlockSpec((1000, 8), lambda r: (r, 0))
        hw_shape = jax.ShapeDtypeStruct((_N, 8), _F32)
    return pl.pallas_call(
        _post_body,
        grid=(10,),
        in_specs=[
            pl.BlockSpec((2, 1000, 128), lambda r: (0, r, 0)),
            pl.BlockSpec((2, 1000, 8), lambda r: (0, r, 0)),
            pl.BlockSpec((1000, _H), lambda r: (r, 0)),
            pl.BlockSpec((_H, _H), lambda r: (0, 0)),
            pl.BlockSpec((1, _H), lambda r: (0, 0)),
            pl.BlockSpec((1, _H), lambda r: (0, 0)),
            pl.BlockSpec((1, _H), lambda r: (0, 0)),
            pl.BlockSpec((_H, Wnext.shape[1]), lambda r: (0, 0)),
        ],
        out_specs=[
            pl.BlockSpec((1000, _H), lambda r: (r, 0)),
            hw_spec,
        ],
        out_shape=[
            jax.ShapeDtypeStruct((_N, _H), _F32),
            hw_shape,
        ],
    )(agg, dacc, h, Wr, bl.reshape(1, _H), g.reshape(1, _H), be.reshape(1, _H),
      Wnext)


def _fin_post_body(agg_ref, dacc_ref, s_ref, b_ref, o_ref):
    a = agg_ref[0] + agg_ref[1]
    deg = dacc_ref[0, :, 0:1] + dacc_ref[1, :, 0:1]
    m = jnp.maximum(deg, 1.0)
    o_ref[...] = a / m + b_ref[...] + s_ref[:, 1:2]


def _fin_post(agg8, dacc, s, bl3):
    return pl.pallas_call(
        _fin_post_body,
        grid=(10,),
        in_specs=[
            pl.BlockSpec((2, 1000, 8), lambda r: (0, r, 0)),
            pl.BlockSpec((2, 1000, 8), lambda r: (0, r, 0)),
            pl.BlockSpec((1000, 8), lambda r: (r, 0)),
            pl.BlockSpec((1, 1), lambda r: (0, 0)),
        ],
        out_specs=pl.BlockSpec((1000, 8), lambda r: (r, 0)),
        out_shape=jax.ShapeDtypeStruct((_N, 8), _F32),
    )(agg8, dacc, s, bl3.reshape(1, 1))


def _sc_mesh():
    return plsc.VectorSubcoreMesh(
        core_axis_name="c", subcore_axis_name="s", num_cores=2, num_subcores=16)


def _pipelined_chunks(tbl, acc, src_v, dst_v, rows2, gsem, ssem, n):
    def g_start(j, x):
        pltpu.async_copy(tbl.at[src_v.at[j]], rows2.at[x], gsem)

    def g_wait(j, x):
        pltpu.make_async_copy(tbl.at[src_v.at[j]], rows2.at[x], gsem).wait()

    def s_start(j, x):
        pltpu.async_copy(rows2.at[x], acc.at[dst_v.at[j]], ssem, add=True)

    def s_wait(j, x):
        pltpu.make_async_copy(rows2.at[x], acc.at[dst_v.at[j]], ssem).wait()

    g_start(0, 0)

    def it(j, carry):
        x = lax.rem(j, 2)
        g_wait(j, x)

        @pl.when(j >= 1)
        def _():
            s_wait(j - 1, 1 - x)

        @pl.when(j + 1 < n)
        def _():
            g_start(j + 1, 1 - x)

        s_start(j, x)
        return carry

    lax.fori_loop(0, n, it, 0)
    s_wait(n - 1, (n - 1) % 2)


def _segsum_feat(table, srci, dsti, zeros, chunks):
    assert chunks % 4 == 0
    stage = chunks // 4

    def stage_loop(tbl, acc, src_v, dst_v, rows4, gsem, ssem, n):
        def g_start(j, x):
            pltpu.async_copy(tbl.at[src_v.at[j]], rows4.at[x], gsem)

        def g_wait(j, x):
            pltpu.make_async_copy(tbl.at[src_v.at[j]], rows4.at[x],
                                  gsem).wait()

        def s_start(j, x):
            pltpu.async_copy(rows4.at[x], acc.at[dst_v.at[j]], ssem, add=True)

        def s_wait(j, x):
            pltpu.make_async_copy(rows4.at[x], acc.at[dst_v.at[j]],
                                  ssem).wait()

        g_start(0, 0)
        g_start(1, 1)

        def it(j, carry):
            x = lax.rem(j, 4)
            g_wait(j, x)
            s_start(j, x)

            @pl.when(j >= 2)
            def _():
                s_wait(j - 2, lax.rem(j - 2, 4))

            @pl.when(j + 2 < n)
            def _():
                g_start(j + 2, lax.rem(j + 2, 4))

            return carry

        lax.fori_loop(0, n, it, 0)
        s_wait(n - 2, (n - 2) % 4)
        s_wait(n - 1, (n - 1) % 4)

    def body(tbl, srci_h, dsti_h, zer, out, src_v, dst_v, rows4, acc, gsem,
             ssem):
        c = lax.axis_index("c")
        s = lax.axis_index("s")
        for k in range(5):
            pltpu.sync_copy(zer, acc.at[pl.ds(s * 640 + k * 128, 128)])
        plsc.subcore_barrier()
        for k in range(4):
            pltpu.sync_copy(srci_h.at[c, s, pl.ds(k * stage, stage)], src_v)
            pltpu.sync_copy(dsti_h.at[s, pl.ds(k * stage, stage)], dst_v)
            stage_loop(tbl, acc, src_v, dst_v, rows4, gsem, ssem, stage)
        plsc.subcore_barrier()
        pltpu.sync_copy(acc.at[pl.ds(s * 640, 640)],
                        out.at[c, pl.ds(s * 640, 640)])

    f = pl.kernel(
        body,
        out_type=jax.ShapeDtypeStruct((2, _NACC, 128), _F32),
        mesh=_sc_mesh(),
        compiler_params=pltpu.CompilerParams(use_tc_tiling_on_sc=False),
        scratch_types=[
            pltpu.VMEM((stage, 64), jnp.int32),
            pltpu.VMEM((stage, 64), jnp.int32),
            pltpu.VMEM((4, 64, 128), _F32),
            pltpu.VMEM_SHARED((_NACC, 128), _F32),
            pltpu.SemaphoreType.DMA,
            pltpu.SemaphoreType.DMA,
        ],
    )
    return f(table, srci, dsti, zeros)


def _segsum_edge(table, srci, dsti, zeros, chunks):
    def body(tbl, srci_h, dsti_h, zer, out, src_v, dst_v, rows2, acc, gsem,
             ssem):
        c = lax.axis_index("c")
        s = lax.axis_index("s")
        w = c * 16 + s
        for k in range(5):
            pltpu.sync_copy(zer, acc.at[pl.ds(s * 640 + k * 128, 128)])
        pltpu.sync_copy(srci_h.at[w], src_v)
        pltpu.sync_copy(dsti_h.at[w], dst_v)
        plsc.subcore_barrier()
        _pipelined_chunks(tbl, acc, src_v, dst_v, rows2, gsem, ssem, chunks)
        plsc.subcore_barrier()
        pltpu.sync_copy(acc.at[pl.ds(s * 640, 640)],
                        out.at[c, pl.ds(s * 640, 640)])

    f = pl.kernel(
        body,
        out_type=jax.ShapeDtypeStruct((2, _NACC, 8), _F32),
        mesh=_sc_mesh(),
        compiler_params=pltpu.CompilerParams(use_tc_tiling_on_sc=False),
        scratch_types=[
            pltpu.VMEM((chunks, 128), jnp.int32),
            pltpu.VMEM((chunks, 128), jnp.int32),
            pltpu.VMEM((2, 128, 8), _F32),
            pltpu.VMEM_SHARED((_NACC, 8), _F32),
            pltpu.SemaphoreType.DMA,
            pltpu.SemaphoreType.DMA,
        ],
    )
    return f(table, srci, dsti, zeros)


def _deg_count(ones, zeros, dsti, chunks):
    def body(one_h, zer, dsti_h, out, dst_v, rows, acc, ssem):
        c = lax.axis_index("c")
        s = lax.axis_index("s")
        w = c * 16 + s
        for k in range(5):
            pltpu.sync_copy(zer, acc.at[pl.ds(s * 640 + k * 128, 128)])
        pltpu.sync_copy(one_h, rows)
        pltpu.sync_copy(dsti_h.at[w], dst_v)
        plsc.subcore_barrier()

        def fire(j, carry):
            pltpu.async_copy(rows, acc.at[dst_v.at[j]], ssem, add=True)
            return carry

        def drain(j, carry):
            pltpu.make_async_copy(rows, acc.at[dst_v.at[j]], ssem).wait()
            return carry

        def grp(k, carry):
            lax.fori_loop(k * 8, k * 8 + 8, fire, 0)
            lax.fori_loop(k * 8, k * 8 + 8, drain, 0)
            return carry

        lax.fori_loop(0, chunks // 8, grp, 0)
        plsc.subcore_barrier()
        pltpu.sync_copy(acc.at[pl.ds(s * 640, 640)],
                        out.at[c, pl.ds(s * 640, 640)])

    f = pl.kernel(
        body,
        out_type=jax.ShapeDtypeStruct((2, _NACC, 8), _F32),
        mesh=_sc_mesh(),
        compiler_params=pltpu.CompilerParams(use_tc_tiling_on_sc=False),
        scratch_types=[
            pltpu.VMEM((chunks, 128), jnp.int32),
            pltpu.VMEM((128, 8), _F32),
            pltpu.VMEM_SHARED((_NACC, 8), _F32),
            pltpu.SemaphoreType.DMA,
        ],
    )
    return f(ones, zeros, dsti)


def kernel(x, edge_index, Wp, bp, Wl0, bl0, Wr0, g0, be0, Wl1, bl1, Wr1, g1,
           be1, Wl2, bl2, Wr2, g2, be2, Wl3, bl3, Wr3):
    src = edge_index[0]
    dst = edge_index[1]
    e = src.shape[0]
    ep = -(-e // 32768) * 32768
    pad = ep - e
    srcp = jnp.concatenate([src, jnp.zeros((pad,), jnp.int32)])
    dstp = jnp.concatenate([dst, jnp.full((pad,), _N, jnp.int32)])
    ch128 = ep // (16 * 64)
    ch8 = ep // (32 * 128)
    src128 = jnp.stack([srcp, srcp + _N]).reshape(2, 16, ch128, 64)
    dst128 = dstp.reshape(16, ch128, 64)
    src8 = srcp.reshape(32, ch8, 128)
    dst8 = dstp.reshape(32, ch8, 128)
    zeros128 = jnp.zeros((128, 128), _F32)
    zeros8 = jnp.zeros((128, 8), _F32)
    ones8 = jnp.ones((128, 8), _F32)

    W3 = jnp.concatenate([Wl3, Wr3, jnp.zeros((_H, 6), _F32)], axis=1)
    dacc = _deg_count(ones8, zeros8, dst8, ch8)
    h, hw = _proj(x, Wp, bp, Wl0)
    for (Wr, bl, g, be, Wnext) in ((Wr0, bl0, g0, be0, Wl1),
                                   (Wr1, bl1, g1, be1, Wl2),
                                   (Wr2, bl2, g2, be2, W3)):
        agg = _segsum_feat(hw.reshape(2 * _N, 128), src128, dst128, zeros128,
                           ch128)
        h, hw = _post(agg, dacc, h, Wr, bl, g, be, Wnext)

    s = hw
    agg8 = _segsum_edge(s, src8, dst8, zeros8, ch8)
    fin = _fin_post(agg8, dacc, s, bl3)
    return fin[:, 0]


# speedup vs baseline: 1.6959x; 1.6959x over previous
import functools

import jax
import jax.numpy as jnp
from jax import lax
from jax.experimental import pallas as pl
from jax.experimental.pallas import tpu as pltpu
from jax.experimental.pallas import tpu_sc as plsc

_N = 10000
_H = 256
_NACC = 10240
_F32 = jnp.float32


def _proj_body(x_ref, w_ref, b_ref, wl_ref, o_ref, hw_ref):
    h = jnp.maximum(
        jnp.dot(x_ref[...], w_ref[...], preferred_element_type=_F32)
        + b_ref[...], 0.0)
    o_ref[...] = h
    hw = jnp.dot(h, wl_ref[...], preferred_element_type=_F32)
    for q in range(4):
        hw_ref[q] = hw[:, q * 64:(q + 1) * 64]


def _proj(x, Wp, bp, Wl0):
    return pl.pallas_call(
        _proj_body,
        grid=(10,),
        in_specs=[
            pl.BlockSpec((1000, 128), lambda i: (i, 0)),
            pl.BlockSpec((128, _H), lambda i: (0, 0)),
            pl.BlockSpec((1, _H), lambda i: (0, 0)),
            pl.BlockSpec((_H, _H), lambda i: (0, 0)),
        ],
        out_specs=[
            pl.BlockSpec((1000, _H), lambda i: (i, 0)),
            pl.BlockSpec((4, 1000, 64), lambda i: (0, i, 0)),
        ],
        out_shape=[
            jax.ShapeDtypeStruct((_N, _H), _F32),
            jax.ShapeDtypeStruct((4, _NACC, 64), _F32),
        ],
    )(x, Wp, bp.reshape(1, _H), Wl0)


def _post_body(agg_ref, dacc_ref, h_ref, wr_ref, bl_ref, g_ref, be_ref,
               wn_ref, o_ref, hw_ref):
    mean_cat = jnp.concatenate([agg_ref[q] for q in range(4)], axis=-1)
    deg = dacc_ref[0, :, 0:1] + dacc_ref[1, :, 0:1]
    m = jnp.maximum(deg, 1.0)
    h = h_ref[...]
    z = mean_cat / m + bl_ref[...] + jnp.dot(
        h, wr_ref[...], preferred_element_type=_F32)
    mu = jnp.mean(z, axis=-1, keepdims=True)
    zc = z - mu
    var = jnp.mean(zc * zc, axis=-1, keepdims=True)
    zn = zc * lax.rsqrt(var + 1e-5) * g_ref[...] + be_ref[...]
    h_new = jnp.maximum(zn, 0.0) + h
    o_ref[...] = h_new
    hw = jnp.dot(h_new, wn_ref[...], preferred_element_type=_F32)
    if hw_ref.shape[0] == 4:
        for q in range(4):
            hw_ref[q] = hw[:, q * 64:(q + 1) * 64]
    else:
        hw_ref[...] = hw


def _post(agg, dacc, h, Wr, bl, g, be, Wnext):
    wide = Wnext.shape[1] == _H
    if wide:
        hw_spec = pl.BlockSpec((4, 1000, 64), lambda r: (0, r, 0))
        hw_shape = jax.ShapeDtypeStruct((4, _NACC, 64), _F32)
    else:
        hw_spec = pl.BlockSpec((1000, 8), lambda r: (r, 0))
        hw_shape = jax.ShapeDtypeStruct((_N, 8), _F32)
    return pl.pallas_call(
        _post_body,
        grid=(10,),
        in_specs=[
            pl.BlockSpec((4, 1000, 64), lambda r: (0, r, 0)),
            pl.BlockSpec((2, 1000, 8), lambda r: (0, r, 0)),
            pl.BlockSpec((1000, _H), lambda r: (r, 0)),
            pl.BlockSpec((_H, _H), lambda r: (0, 0)),
            pl.BlockSpec((1, _H), lambda r: (0, 0)),
            pl.BlockSpec((1, _H), lambda r: (0, 0)),
            pl.BlockSpec((1, _H), lambda r: (0, 0)),
            pl.BlockSpec((_H, Wnext.shape[1]), lambda r: (0, 0)),
        ],
        out_specs=[
            pl.BlockSpec((1000, _H), lambda r: (r, 0)),
            hw_spec,
        ],
        out_shape=[
            jax.ShapeDtypeStruct((_N, _H), _F32),
            hw_shape,
        ],
    )(agg, dacc, h, Wr, bl.reshape(1, _H), g.reshape(1, _H), be.reshape(1, _H),
      Wnext)


def _fin_post_body(agg_ref, dacc_ref, s_ref, b_ref, o_ref):
    a = agg_ref[0] + agg_ref[1]
    deg = dacc_ref[0, :, 0:1] + dacc_ref[1, :, 0:1]
    m = jnp.maximum(deg, 1.0)
    o_ref[...] = a / m + b_ref[...] + s_ref[:, 1:2]


def _fin_post(agg8, dacc, s, bl3):
    return pl.pallas_call(
        _fin_post_body,
        grid=(10,),
        in_specs=[
            pl.BlockSpec((2, 1000, 8), lambda r: (0, r, 0)),
            pl.BlockSpec((2, 1000, 8), lambda r: (0, r, 0)),
            pl.BlockSpec((1000, 8), lambda r: (r, 0)),
            pl.BlockSpec((1, 1), lambda r: (0, 0)),
        ],
        out_specs=pl.BlockSpec((1000, 8), lambda r: (r, 0)),
        out_shape=jax.ShapeDtypeStruct((_N, 8), _F32),
    )(agg8, dacc, s, bl3.reshape(1, 1))


def _sc_mesh():
    return plsc.VectorSubcoreMesh(
        core_axis_name="c", subcore_axis_name="s", num_cores=2, num_subcores=16)


def _pipelined_chunks(tbl, acc, src_v, dst_v, rows2, gsem, ssem, n):
    def g_start(j, x):
        pltpu.async_copy(tbl.at[src_v.at[j]], rows2.at[x], gsem)

    def g_wait(j, x):
        pltpu.make_async_copy(tbl.at[src_v.at[j]], rows2.at[x], gsem).wait()

    def s_start(j, x):
        pltpu.async_copy(rows2.at[x], acc.at[dst_v.at[j]], ssem, add=True)

    def s_wait(j, x):
        pltpu.make_async_copy(rows2.at[x], acc.at[dst_v.at[j]], ssem).wait()

    g_start(0, 0)

    def it(j, carry):
        x = lax.rem(j, 2)
        g_wait(j, x)

        @pl.when(j >= 1)
        def _():
            s_wait(j - 1, 1 - x)

        @pl.when(j + 1 < n)
        def _():
            g_start(j + 1, 1 - x)

        s_start(j, x)
        return carry

    lax.fori_loop(0, n, it, 0)
    s_wait(n - 1, (n - 1) % 2)


def _segsum_feat(table, srci, dsti, zeros, chunks):
    assert chunks % 4 == 0
    stage = chunks // 4

    def stage_loop(tbl, acc, src_v, dst_v, rows4, gsem, ssem, n):
        def g_start(j, x):
            pltpu.async_copy(tbl.at[src_v.at[j]], rows4.at[x], gsem)

        def g_wait(j, x):
            pltpu.make_async_copy(tbl.at[src_v.at[j]], rows4.at[x],
                                  gsem).wait()

        def s_start(j, x):
            pltpu.async_copy(rows4.at[x], acc.at[dst_v.at[j]], ssem, add=True)

        def s_wait(j, x):
            pltpu.make_async_copy(rows4.at[x], acc.at[dst_v.at[j]],
                                  ssem).wait()

        g_start(0, 0)
        g_start(1, 1)

        def it(j, carry):
            x = lax.rem(j, 4)
            g_wait(j, x)
            s_start(j, x)

            @pl.when(j >= 2)
            def _():
                s_wait(j - 2, lax.rem(j - 2, 4))

            @pl.when(j + 2 < n)
            def _():
                g_start(j + 2, lax.rem(j + 2, 4))

            return carry

        lax.fori_loop(0, n, it, 0)
        s_wait(n - 2, (n - 2) % 4)
        s_wait(n - 1, (n - 1) % 4)

    def body(tbl_h, srci_h, dsti_h, zer, out, src_v, dst_v, rows4, tblbuf,
             acc, gsem, ssem):
        c = lax.axis_index("c")
        s = lax.axis_index("s")
        for p in range(2):
            q = c * 2 + p
            pltpu.sync_copy(tbl_h.at[pl.ds(q * _NACC + s * 640, 640)],
                            tblbuf.at[pl.ds(s * 640, 640)])
            for k in range(5):
                pltpu.sync_copy(zer, acc.at[pl.ds(s * 640 + k * 128, 128)])
            plsc.subcore_barrier()
            for k in range(4):
                pltpu.sync_copy(srci_h.at[s, pl.ds(k * stage, stage)], src_v)
                pltpu.sync_copy(dsti_h.at[s, pl.ds(k * stage, stage)], dst_v)
                stage_loop(tblbuf, acc, src_v, dst_v, rows4, gsem, ssem,
                           stage)
            plsc.subcore_barrier()
            pltpu.sync_copy(acc.at[pl.ds(s * 640, 640)],
                            out.at[q, pl.ds(s * 640, 640)])

    f = pl.kernel(
        body,
        out_type=jax.ShapeDtypeStruct((4, _NACC, 64), _F32),
        mesh=_sc_mesh(),
        compiler_params=pltpu.CompilerParams(use_tc_tiling_on_sc=False),
        scratch_types=[
            pltpu.VMEM((stage, 64), jnp.int32),
            pltpu.VMEM((stage, 64), jnp.int32),
            pltpu.VMEM((4, 64, 64), _F32),
            pltpu.VMEM_SHARED((_NACC, 64), _F32),
            pltpu.VMEM_SHARED((_NACC, 64), _F32),
            pltpu.SemaphoreType.DMA,
            pltpu.SemaphoreType.DMA,
        ],
    )
    return f(table, srci, dsti, zeros)


def _segsum_edge(table, srci, dsti, zeros, chunks):
    def body(tbl, srci_h, dsti_h, zer, out, src_v, dst_v, rows2, acc, gsem,
             ssem):
        c = lax.axis_index("c")
        s = lax.axis_index("s")
        w = c * 16 + s
        for k in range(5):
            pltpu.sync_copy(zer, acc.at[pl.ds(s * 640 + k * 128, 128)])
        pltpu.sync_copy(srci_h.at[w], src_v)
        pltpu.sync_copy(dsti_h.at[w], dst_v)
        plsc.subcore_barrier()
        _pipelined_chunks(tbl, acc, src_v, dst_v, rows2, gsem, ssem, chunks)
        plsc.subcore_barrier()
        pltpu.sync_copy(acc.at[pl.ds(s * 640, 640)],
                        out.at[c, pl.ds(s * 640, 640)])

    f = pl.kernel(
        body,
        out_type=jax.ShapeDtypeStruct((2, _NACC, 8), _F32),
        mesh=_sc_mesh(),
        compiler_params=pltpu.CompilerParams(use_tc_tiling_on_sc=False),
        scratch_types=[
            pltpu.VMEM((chunks, 128), jnp.int32),
            pltpu.VMEM((chunks, 128), jnp.int32),
            pltpu.VMEM((2, 128, 8), _F32),
            pltpu.VMEM_SHARED((_NACC, 8), _F32),
            pltpu.SemaphoreType.DMA,
            pltpu.SemaphoreType.DMA,
        ],
    )
    return f(table, srci, dsti, zeros)


def _deg_count(ones, zeros, dsti, chunks):
    def body(one_h, zer, dsti_h, out, dst_v, rows, acc, ssem):
        c = lax.axis_index("c")
        s = lax.axis_index("s")
        w = c * 16 + s
        for k in range(5):
            pltpu.sync_copy(zer, acc.at[pl.ds(s * 640 + k * 128, 128)])
        pltpu.sync_copy(one_h, rows)
        pltpu.sync_copy(dsti_h.at[w], dst_v)
        plsc.subcore_barrier()

        def fire(j, carry):
            pltpu.async_copy(rows, acc.at[dst_v.at[j]], ssem, add=True)
            return carry

        def drain(j, carry):
            pltpu.make_async_copy(rows, acc.at[dst_v.at[j]], ssem).wait()
            return carry

        def grp(k, carry):
            lax.fori_loop(k * 8, k * 8 + 8, fire, 0)
            lax.fori_loop(k * 8, k * 8 + 8, drain, 0)
            return carry

        lax.fori_loop(0, chunks // 8, grp, 0)
        plsc.subcore_barrier()
        pltpu.sync_copy(acc.at[pl.ds(s * 640, 640)],
                        out.at[c, pl.ds(s * 640, 640)])

    f = pl.kernel(
        body,
        out_type=jax.ShapeDtypeStruct((2, _NACC, 8), _F32),
        mesh=_sc_mesh(),
        compiler_params=pltpu.CompilerParams(use_tc_tiling_on_sc=False),
        scratch_types=[
            pltpu.VMEM((chunks, 128), jnp.int32),
            pltpu.VMEM((128, 8), _F32),
            pltpu.VMEM_SHARED((_NACC, 8), _F32),
            pltpu.SemaphoreType.DMA,
        ],
    )
    return f(ones, zeros, dsti)


def kernel(x, edge_index, Wp, bp, Wl0, bl0, Wr0, g0, be0, Wl1, bl1, Wr1, g1,
           be1, Wl2, bl2, Wr2, g2, be2, Wl3, bl3, Wr3):
    src = edge_index[0]
    dst = edge_index[1]
    e = src.shape[0]
    ep = -(-e // 32768) * 32768
    pad = ep - e
    srcp = jnp.concatenate([src, jnp.zeros((pad,), jnp.int32)])
    dstp = jnp.concatenate([dst, jnp.full((pad,), _N, jnp.int32)])
    ch128 = ep // (16 * 64)
    ch8 = ep // (32 * 128)
    srcq = srcp.reshape(16, ch128, 64)
    dstq = dstp.reshape(16, ch128, 64)
    src8 = srcp.reshape(32, ch8, 128)
    dst8 = dstp.reshape(32, ch8, 128)
    zeros64 = jnp.zeros((128, 64), _F32)
    zeros8 = jnp.zeros((128, 8), _F32)
    ones8 = jnp.ones((128, 8), _F32)

    W3 = jnp.concatenate([Wl3, Wr3, jnp.zeros((_H, 6), _F32)], axis=1)
    dacc = _deg_count(ones8, zeros8, dst8, ch8)
    h, hw = _proj(x, Wp, bp, Wl0)
    for (Wr, bl, g, be, Wnext) in ((Wr0, bl0, g0, be0, Wl1),
                                   (Wr1, bl1, g1, be1, Wl2),
                                   (Wr2, bl2, g2, be2, W3)):
        agg = _segsum_feat(hw.reshape(4 * _NACC, 64), srcq, dstq, zeros64,
                           ch128)
        h, hw = _post(agg, dacc, h, Wr, bl, g, be, Wnext)

    s = hw
    agg8 = _segsum_edge(s, src8, dst8, zeros8, ch8)
    fin = _fin_post(agg8, dacc, s, bl3)
    return fin[:, 0]


# speedup vs baseline: 1.7862x; 1.0533x over previous
import functools

import jax
import jax.numpy as jnp
from jax import lax
from jax.experimental import pallas as pl
from jax.experimental.pallas import tpu as pltpu
from jax.experimental.pallas import tpu_sc as plsc

_N = 10000
_H = 256
_NACC = 10240
_F32 = jnp.float32


def _proj_body(x_ref, w_ref, b_ref, wl_ref, o_ref, hw_ref):
    h = jnp.maximum(
        jnp.dot(x_ref[...], w_ref[...], preferred_element_type=_F32)
        + b_ref[...], 0.0)
    o_ref[...] = h
    hw = jnp.dot(h, wl_ref[...], preferred_element_type=_F32)
    for q in range(4):
        hw_ref[q] = hw[:, q * 64:(q + 1) * 64]


def _proj(x, Wp, bp, Wl0):
    return pl.pallas_call(
        _proj_body,
        grid=(10,),
        in_specs=[
            pl.BlockSpec((1000, 128), lambda i: (i, 0)),
            pl.BlockSpec((128, _H), lambda i: (0, 0)),
            pl.BlockSpec((1, _H), lambda i: (0, 0)),
            pl.BlockSpec((_H, _H), lambda i: (0, 0)),
        ],
        out_specs=[
            pl.BlockSpec((1000, _H), lambda i: (i, 0)),
            pl.BlockSpec((4, 1000, 64), lambda i: (0, i, 0)),
        ],
        out_shape=[
            jax.ShapeDtypeStruct((_N, _H), _F32),
            jax.ShapeDtypeStruct((4, _NACC, 64), _F32),
        ],
    )(x, Wp, bp.reshape(1, _H), Wl0)


def _post_body(agg_ref, dacc_ref, h_ref, wr_ref, bl_ref, g_ref, be_ref,
               wn_ref, o_ref, hw_ref):
    mean_cat = jnp.concatenate([agg_ref[q] for q in range(4)], axis=-1)
    deg = dacc_ref[0, :, 0:1] + dacc_ref[1, :, 0:1]
    m = jnp.maximum(deg, 1.0)
    h = h_ref[...]
    z = mean_cat / m + bl_ref[...] + jnp.dot(
        h, wr_ref[...], preferred_element_type=_F32)
    mu = jnp.mean(z, axis=-1, keepdims=True)
    zc = z - mu
    var = jnp.mean(zc * zc, axis=-1, keepdims=True)
    zn = zc * lax.rsqrt(var + 1e-5) * g_ref[...] + be_ref[...]
    h_new = jnp.maximum(zn, 0.0) + h
    o_ref[...] = h_new
    hw = jnp.dot(h_new, wn_ref[...], preferred_element_type=_F32)
    if hw_ref.shape[0] == 4:
        for q in range(4):
            hw_ref[q] = hw[:, q * 64:(q + 1) * 64]
    else:
        hw_ref[...] = hw


def _post(agg, dacc, h, Wr, bl, g, be, Wnext):
    wide = Wnext.shape[1] == _H
    if wide:
        hw_spec = pl.BlockSpec((4, 1000, 64), lambda r: (0, r, 0))
        hw_shape = jax.ShapeDtypeStruct((4, _NACC, 64), _F32)
    else:
        hw_spec = pl.BlockSpec((1000, 8), lambda r: (r, 0))
        hw_shape = jax.ShapeDtypeStruct((_NACC, 8), _F32)
    return pl.pallas_call(
        _post_body,
        grid=(10,),
        in_specs=[
            pl.BlockSpec((4, 1000, 64), lambda r: (0, r, 0)),
            pl.BlockSpec((2, 1000, 8), lambda r: (0, r, 0)),
            pl.BlockSpec((1000, _H), lambda r: (r, 0)),
            pl.BlockSpec((_H, _H), lambda r: (0, 0)),
            pl.BlockSpec((1, _H), lambda r: (0, 0)),
            pl.BlockSpec((1, _H), lambda r: (0, 0)),
            pl.BlockSpec((1, _H), lambda r: (0, 0)),
            pl.BlockSpec((_H, Wnext.shape[1]), lambda r: (0, 0)),
        ],
        out_specs=[
            pl.BlockSpec((1000, _H), lambda r: (r, 0)),
            hw_spec,
        ],
        out_shape=[
            jax.ShapeDtypeStruct((_N, _H), _F32),
            hw_shape,
        ],
    )(agg, dacc, h, Wr, bl.reshape(1, _H), g.reshape(1, _H), be.reshape(1, _H),
      Wnext)


def _fin_post_body(agg_ref, dacc_ref, s_ref, b_ref, o_ref):
    a = agg_ref[0] + agg_ref[1]
    deg = dacc_ref[0, :, 0:1] + dacc_ref[1, :, 0:1]
    m = jnp.maximum(deg, 1.0)
    o_ref[...] = a / m + b_ref[...] + s_ref[:, 1:2]


def _fin_post(agg8, dacc, s, bl3):
    return pl.pallas_call(
        _fin_post_body,
        grid=(10,),
        in_specs=[
            pl.BlockSpec((2, 1000, 8), lambda r: (0, r, 0)),
            pl.BlockSpec((2, 1000, 8), lambda r: (0, r, 0)),
            pl.BlockSpec((1000, 8), lambda r: (r, 0)),
            pl.BlockSpec((1, 1), lambda r: (0, 0)),
        ],
        out_specs=pl.BlockSpec((1000, 8), lambda r: (r, 0)),
        out_shape=jax.ShapeDtypeStruct((_N, 8), _F32),
    )(agg8, dacc, s, bl3.reshape(1, 1))


def _sc_mesh():
    return plsc.VectorSubcoreMesh(
        core_axis_name="c", subcore_axis_name="s", num_cores=2, num_subcores=16)


def _pipelined_chunks(tbl, acc, src_v, dst_v, rows2, gsem, ssem, n):
    def g_start(j, x):
        pltpu.async_copy(tbl.at[src_v.at[j]], rows2.at[x], gsem)

    def g_wait(j, x):
        pltpu.make_async_copy(tbl.at[src_v.at[j]], rows2.at[x], gsem).wait()

    def s_start(j, x):
        pltpu.async_copy(rows2.at[x], acc.at[dst_v.at[j]], ssem, add=True)

    def s_wait(j, x):
        pltpu.make_async_copy(rows2.at[x], acc.at[dst_v.at[j]], ssem).wait()

    g_start(0, 0)

    def it(j, carry):
        x = lax.rem(j, 2)
        g_wait(j, x)

        @pl.when(j >= 1)
        def _():
            s_wait(j - 1, 1 - x)

        @pl.when(j + 1 < n)
        def _():
            g_start(j + 1, 1 - x)

        s_start(j, x)
        return carry

    lax.fori_loop(0, n, it, 0)
    s_wait(n - 1, (n - 1) % 2)


def _segsum_feat(table, srci, dsti, zeros, chunks):
    assert chunks % 4 == 0
    stage = chunks // 4

    def stage_loop(tbl, acc, src_v, dst_v, rows4, gsem, ssem, n):
        def g_start(j, x):
            pltpu.async_copy(tbl.at[src_v.at[j]], rows4.at[x], gsem)

        def g_wait(j, x):
            pltpu.make_async_copy(tbl.at[src_v.at[j]], rows4.at[x],
                                  gsem).wait()

        def s_start(j, x):
            pltpu.async_copy(rows4.at[x], acc.at[dst_v.at[j]], ssem, add=True)

        def s_wait(j, x):
            pltpu.make_async_copy(rows4.at[x], acc.at[dst_v.at[j]],
                                  ssem).wait()

        g_start(0, 0)
        g_start(1, 1)

        def it(j, carry):
            x = lax.rem(j, 4)
            g_wait(j, x)
            s_start(j, x)

            @pl.when(j >= 2)
            def _():
                s_wait(j - 2, lax.rem(j - 2, 4))

            @pl.when(j + 2 < n)
            def _():
                g_start(j + 2, lax.rem(j + 2, 4))

            return carry

        lax.fori_loop(0, n, it, 0)
        s_wait(n - 2, (n - 2) % 4)
        s_wait(n - 1, (n - 1) % 4)

    def body(tbl_h, srci_h, dsti_h, zer, out, src_v, dst_v, rows4, tblbuf,
             acc, gsem, ssem):
        c = lax.axis_index("c")
        s = lax.axis_index("s")
        for p in range(2):
            q = c * 2 + p
            pltpu.sync_copy(tbl_h.at[pl.ds(q * _NACC + s * 640, 640)],
                            tblbuf.at[pl.ds(s * 640, 640)])
            for k in range(5):
                pltpu.sync_copy(zer, acc.at[pl.ds(s * 640 + k * 128, 128)])
            plsc.subcore_barrier()
            for k in range(4):
                pltpu.sync_copy(srci_h.at[s, pl.ds(k * stage, stage)], src_v)
                pltpu.sync_copy(dsti_h.at[s, pl.ds(k * stage, stage)], dst_v)
                stage_loop(tblbuf, acc, src_v, dst_v, rows4, gsem, ssem,
                           stage)
            plsc.subcore_barrier()
            pltpu.sync_copy(acc.at[pl.ds(s * 640, 640)],
                            out.at[q, pl.ds(s * 640, 640)])

    f = pl.kernel(
        body,
        out_type=jax.ShapeDtypeStruct((4, _NACC, 64), _F32),
        mesh=_sc_mesh(),
        compiler_params=pltpu.CompilerParams(use_tc_tiling_on_sc=False),
        scratch_types=[
            pltpu.VMEM((stage, 64), jnp.int32),
            pltpu.VMEM((stage, 64), jnp.int32),
            pltpu.VMEM((4, 64, 64), _F32),
            pltpu.VMEM_SHARED((_NACC, 64), _F32),
            pltpu.VMEM_SHARED((_NACC, 64), _F32),
            pltpu.SemaphoreType.DMA,
            pltpu.SemaphoreType.DMA,
        ],
    )
    return f(table, srci, dsti, zeros)


def _segsum_edge(table, srci, dsti, zeros, chunks):
    def body(tbl_h, srci_h, dsti_h, zer, out, src_v, dst_v, rows2, tblbuf,
             acc, gsem, ssem):
        c = lax.axis_index("c")
        s = lax.axis_index("s")
        w = c * 16 + s
        pltpu.sync_copy(tbl_h.at[pl.ds(s * 640, 640)],
                        tblbuf.at[pl.ds(s * 640, 640)])
        for k in range(5):
            pltpu.sync_copy(zer, acc.at[pl.ds(s * 640 + k * 128, 128)])
        pltpu.sync_copy(srci_h.at[w], src_v)
        pltpu.sync_copy(dsti_h.at[w], dst_v)
        plsc.subcore_barrier()
        _pipelined_chunks(tblbuf, acc, src_v, dst_v, rows2, gsem, ssem,
                          chunks)
        plsc.subcore_barrier()
        pltpu.sync_copy(acc.at[pl.ds(s * 640, 640)],
                        out.at[c, pl.ds(s * 640, 640)])

    f = pl.kernel(
        body,
        out_type=jax.ShapeDtypeStruct((2, _NACC, 8), _F32),
        mesh=_sc_mesh(),
        compiler_params=pltpu.CompilerParams(use_tc_tiling_on_sc=False),
        scratch_types=[
            pltpu.VMEM((chunks, 128), jnp.int32),
            pltpu.VMEM((chunks, 128), jnp.int32),
            pltpu.VMEM((2, 128, 8), _F32),
            pltpu.VMEM_SHARED((_NACC, 8), _F32),
            pltpu.VMEM_SHARED((_NACC, 8), _F32),
            pltpu.SemaphoreType.DMA,
            pltpu.SemaphoreType.DMA,
        ],
    )
    return f(table, srci, dsti, zeros)


def _deg_count(ones, zeros, dsti, chunks):
    def body(one_h, zer, dsti_h, out, dst_v, rows, acc, ssem):
        c = lax.axis_index("c")
        s = lax.axis_index("s")
        w = c * 16 + s
        for k in range(5):
            pltpu.sync_copy(zer, acc.at[pl.ds(s * 640 + k * 128, 128)])
        pltpu.sync_copy(one_h, rows)
        pltpu.sync_copy(dsti_h.at[w], dst_v)
        plsc.subcore_barrier()

        def fire(j, carry):
            pltpu.async_copy(rows, acc.at[dst_v.at[j]], ssem, add=True)
            return carry

        def drain(j, carry):
            pltpu.make_async_copy(rows, acc.at[dst_v.at[j]], ssem).wait()
            return carry

        def grp(k, carry):
            lax.fori_loop(k * 8, k * 8 + 8, fire, 0)
            lax.fori_loop(k * 8, k * 8 + 8, drain, 0)
            return carry

        lax.fori_loop(0, chunks // 8, grp, 0)
        plsc.subcore_barrier()
        pltpu.sync_copy(acc.at[pl.ds(s * 640, 640)],
                        out.at[c, pl.ds(s * 640, 640)])

    f = pl.kernel(
        body,
        out_type=jax.ShapeDtypeStruct((2, _NACC, 8), _F32),
        mesh=_sc_mesh(),
        compiler_params=pltpu.CompilerParams(use_tc_tiling_on_sc=False),
        scratch_types=[
            pltpu.VMEM((chunks, 128), jnp.int32),
            pltpu.VMEM((128, 8), _F32),
            pltpu.VMEM_SHARED((_NACC, 8), _F32),
            pltpu.SemaphoreType.DMA,
        ],
    )
    return f(ones, zeros, dsti)


def kernel(x, edge_index, Wp, bp, Wl0, bl0, Wr0, g0, be0, Wl1, bl1, Wr1, g1,
           be1, Wl2, bl2, Wr2, g2, be2, Wl3, bl3, Wr3):
    src = edge_index[0]
    dst = edge_index[1]
    e = src.shape[0]
    ep = -(-e // 32768) * 32768
    pad = ep - e
    srcp = jnp.concatenate([src, jnp.zeros((pad,), jnp.int32)])
    dstp = jnp.concatenate([dst, jnp.full((pad,), _N, jnp.int32)])
    ch128 = ep // (16 * 64)
    ch8 = ep // (32 * 128)
    srcq = srcp.reshape(16, ch128, 64)
    dstq = dstp.reshape(16, ch128, 64)
    src8 = srcp.reshape(32, ch8, 128)
    dst8 = dstp.reshape(32, ch8, 128)
    zeros64 = jnp.zeros((128, 64), _F32)
    zeros8 = jnp.zeros((128, 8), _F32)
    ones8 = jnp.ones((128, 8), _F32)

    W3 = jnp.concatenate([Wl3, Wr3, jnp.zeros((_H, 6), _F32)], axis=1)
    dacc = _deg_count(ones8, zeros8, dst8, ch8)
    h, hw = _proj(x, Wp, bp, Wl0)
    for (Wr, bl, g, be, Wnext) in ((Wr0, bl0, g0, be0, Wl1),
                                   (Wr1, bl1, g1, be1, Wl2),
                                   (Wr2, bl2, g2, be2, W3)):
        agg = _segsum_feat(hw.reshape(4 * _NACC, 64), srcq, dstq, zeros64,
                           ch128)
        h, hw = _post(agg, dacc, h, Wr, bl, g, be, Wnext)

    s = hw
    agg8 = _segsum_edge(s, src8, dst8, zeros8, ch8)
    fin = _fin_post(agg8, dacc, s, bl3)
    return fin[:, 0]
